# baseline (device time: 44935 ns/iter reference)
import jax
import jax.numpy as jnp
from jax import lax
from jax.experimental import pallas as pl
from jax.experimental.pallas import tpu as pltpu


def kernel(ids, E):
    v_local, d = E.shape
    (t_total,) = ids.shape

    my_x = lax.axis_index("x")
    local = ids - my_x * v_local
    mine = (local >= 0) & (local < v_local)
    tok = jnp.nonzero(mine, size=t_total, fill_value=0)[0].astype(jnp.int32)
    row = jnp.where(mine, local, 0)[tok].astype(jnp.int32)
    k = jnp.sum(mine).astype(jnp.int32).reshape(1)

    def body(k_ref, tok_ref, row_ref, e_ref, out_ref, copy_sem, send_sem, recv_sem):
        x = lax.axis_index("x")
        y = lax.axis_index("y")
        z = lax.axis_index("z")
        partner = (1 - x, y, z)
        my_k = k_ref[0]

        barrier_sem = pltpu.get_barrier_semaphore()
        pl.semaphore_signal(
            barrier_sem, inc=1,
            device_id=partner, device_id_type=pl.DeviceIdType.MESH,
        )
        pl.semaphore_wait(barrier_sem, 1)

        def step(i, c):
            t = tok_ref[i]
            r = row_ref[i]
            src = e_ref.at[pl.ds(r, 1), :]
            pltpu.make_async_copy(
                src, out_ref.at[pl.ds(t, 1), :], copy_sem
            ).start()
            pltpu.make_async_remote_copy(
                src_ref=src,
                dst_ref=out_ref.at[pl.ds(t, 1), :],
                send_sem=send_sem,
                recv_sem=recv_sem,
                device_id=partner,
                device_id_type=pl.DeviceIdType.MESH,
            ).start()
            return c

        lax.fori_loop(0, my_k, step, 0)

        def drain_mine(_, c):
            pltpu.make_async_copy(
                e_ref.at[pl.ds(0, 1), :], out_ref.at[pl.ds(0, 1), :], copy_sem
            ).wait()
            pltpu.make_async_remote_copy(
                src_ref=e_ref.at[pl.ds(0, 1), :],
                dst_ref=out_ref.at[pl.ds(0, 1), :],
                send_sem=send_sem,
                recv_sem=recv_sem,
                device_id=partner,
                device_id_type=pl.DeviceIdType.MESH,
            ).wait_send()
            return c

        def drain_recv(_, c):
            pltpu.make_async_remote_copy(
                src_ref=e_ref.at[pl.ds(0, 1), :],
                dst_ref=out_ref.at[pl.ds(0, 1), :],
                send_sem=send_sem,
                recv_sem=recv_sem,
                device_id=partner,
                device_id_type=pl.DeviceIdType.MESH,
            ).wait_recv()
            return c

        lax.fori_loop(0, my_k, drain_mine, 0)
        lax.fori_loop(0, t_total - my_k, drain_recv, 0)

    return pl.pallas_call(
        body,
        out_shape=jax.ShapeDtypeStruct((t_total, d), jnp.float32),
        in_specs=[
            pl.BlockSpec(memory_space=pltpu.SMEM),
            pl.BlockSpec(memory_space=pltpu.SMEM),
            pl.BlockSpec(memory_space=pltpu.SMEM),
            pl.BlockSpec(memory_space=pl.ANY),
        ],
        out_specs=pl.BlockSpec(memory_space=pltpu.VMEM),
        scratch_shapes=[
            pltpu.SemaphoreType.DMA,
            pltpu.SemaphoreType.DMA,
            pltpu.SemaphoreType.DMA,
        ],
        compiler_params=pltpu.CompilerParams(collective_id=0),
    )(k, tok, row, E)


# device time: 40381 ns/iter; 1.1128x vs baseline; 1.1128x over previous
import jax
import jax.numpy as jnp
from jax import lax
from jax.experimental import pallas as pl
from jax.experimental.pallas import tpu as pltpu


def kernel(ids, E):
    v_local, d = E.shape
    (t_total,) = ids.shape

    def body(ids_ref, e_ref, out_ref, tok_ref, row_ref,
             copy_sem, send_sem, recv_sem):
        x = lax.axis_index("x")
        y = lax.axis_index("y")
        z = lax.axis_index("z")
        partner = (1 - x, y, z)
        lo = x * v_local

        barrier_sem = pltpu.get_barrier_semaphore()
        pl.semaphore_signal(
            barrier_sem, inc=1,
            device_id=partner, device_id_type=pl.DeviceIdType.MESH,
        )
        pl.semaphore_wait(barrier_sem, 1)

        def comp(t, c):
            r = ids_ref[t] - lo
            mine = (r >= 0) & (r < v_local)

            @pl.when(mine)
            def _():
                tok_ref[c] = t
                row_ref[c] = r

            return c + jnp.where(mine, 1, 0).astype(jnp.int32)

        k = lax.fori_loop(0, t_total, comp, jnp.int32(0))

        def step(i, c):
            t = tok_ref[i]
            src = e_ref.at[pl.ds(row_ref[i], 1), :]
            dst = out_ref.at[pl.ds(t, 1), :]
            pltpu.make_async_copy(src, dst, copy_sem).start()
            pltpu.make_async_remote_copy(
                src_ref=src,
                dst_ref=dst,
                send_sem=send_sem,
                recv_sem=recv_sem,
                device_id=partner,
                device_id_type=pl.DeviceIdType.MESH,
            ).start()
            return c

        lax.fori_loop(0, k, step, 0)

        def drain_mine(_, c):
            pltpu.make_async_copy(
                e_ref.at[pl.ds(0, 1), :], out_ref.at[pl.ds(0, 1), :], copy_sem
            ).wait()
            pltpu.make_async_remote_copy(
                src_ref=e_ref.at[pl.ds(0, 1), :],
                dst_ref=out_ref.at[pl.ds(0, 1), :],
                send_sem=send_sem,
                recv_sem=recv_sem,
                device_id=partner,
                device_id_type=pl.DeviceIdType.MESH,
            ).wait_send()
            return c

        def drain_recv(_, c):
            pltpu.make_async_remote_copy(
                src_ref=e_ref.at[pl.ds(0, 1), :],
                dst_ref=out_ref.at[pl.ds(0, 1), :],
                send_sem=send_sem,
                recv_sem=recv_sem,
                device_id=partner,
                device_id_type=pl.DeviceIdType.MESH,
            ).wait_recv()
            return c

        lax.fori_loop(0, k, drain_mine, 0)
        lax.fori_loop(0, t_total - k, drain_recv, 0)

    return pl.pallas_call(
        body,
        out_shape=jax.ShapeDtypeStruct((t_total, d), jnp.float32),
        in_specs=[
            pl.BlockSpec(memory_space=pltpu.SMEM),
            pl.BlockSpec(memory_space=pl.ANY),
        ],
        out_specs=pl.BlockSpec(memory_space=pltpu.VMEM),
        scratch_shapes=[
            pltpu.SMEM((t_total,), jnp.int32),
            pltpu.SMEM((t_total,), jnp.int32),
            pltpu.SemaphoreType.DMA,
            pltpu.SemaphoreType.DMA,
            pltpu.SemaphoreType.DMA,
        ],
        compiler_params=pltpu.CompilerParams(collective_id=0),
    )(ids, E)


# device time: 30634 ns/iter; 1.4668x vs baseline; 1.3182x over previous
import jax
import jax.numpy as jnp
from jax import lax
from jax.experimental import pallas as pl
from jax.experimental.pallas import tpu as pltpu


def kernel(ids, E):
    v_local, d = E.shape
    (t_total,) = ids.shape

    def body(ids_ref, e_ref, out_ref, tok_ref, row_ref,
             copy_sem, send_sem, recv_sem):
        x = lax.axis_index("x")
        y = lax.axis_index("y")
        z = lax.axis_index("z")
        partner = (1 - x, y, z)
        lo = x * v_local

        barrier_sem = pltpu.get_barrier_semaphore()
        pl.semaphore_signal(
            barrier_sem, inc=1,
            device_id=partner, device_id_type=pl.DeviceIdType.MESH,
        )
        pl.semaphore_wait(barrier_sem, 1)

        blk = 128
        n_blk = t_total // blk

        def comp(t, c):
            r = ids_ref[t] - lo
            mine = (r >= 0) & (r < v_local)
            tok_ref[c] = t
            row_ref[c] = r
            return c + mine.astype(jnp.int32)

        def issue(i, c):
            t = tok_ref[i]
            src = e_ref.at[pl.ds(row_ref[i], 1), :]
            dst = out_ref.at[pl.ds(t, 1), :]
            pltpu.make_async_remote_copy(
                src_ref=src,
                dst_ref=dst,
                send_sem=send_sem,
                recv_sem=recv_sem,
                device_id=partner,
                device_id_type=pl.DeviceIdType.MESH,
            ).start()
            pltpu.make_async_copy(src, dst, copy_sem).start()
            return c

        def block(b, c0):
            c1 = lax.fori_loop(b * blk, (b + 1) * blk, comp, c0)
            lax.fori_loop(c0, c1, issue, jnp.int32(0))
            return c1

        k = lax.fori_loop(0, n_blk, block, jnp.int32(0))

        def drain_mine(_, c):
            pltpu.make_async_copy(
                e_ref.at[pl.ds(0, 1), :], out_ref.at[pl.ds(0, 1), :], copy_sem
            ).wait()
            pltpu.make_async_remote_copy(
                src_ref=e_ref.at[pl.ds(0, 1), :],
                dst_ref=out_ref.at[pl.ds(0, 1), :],
                send_sem=send_sem,
                recv_sem=recv_sem,
                device_id=partner,
                device_id_type=pl.DeviceIdType.MESH,
            ).wait_send()
            return c

        def drain_recv(_, c):
            pltpu.make_async_remote_copy(
                src_ref=e_ref.at[pl.ds(0, 1), :],
                dst_ref=out_ref.at[pl.ds(0, 1), :],
                send_sem=send_sem,
                recv_sem=recv_sem,
                device_id=partner,
                device_id_type=pl.DeviceIdType.MESH,
            ).wait_recv()
            return c

        lax.fori_loop(0, k, drain_mine, 0)
        lax.fori_loop(0, t_total - k, drain_recv, 0)

    return pl.pallas_call(
        body,
        out_shape=jax.ShapeDtypeStruct((t_total, d), jnp.float32),
        in_specs=[
            pl.BlockSpec(memory_space=pltpu.SMEM),
            pl.BlockSpec(memory_space=pl.ANY),
        ],
        out_specs=pl.BlockSpec(memory_space=pltpu.VMEM),
        scratch_shapes=[
            pltpu.SMEM((t_total,), jnp.int32),
            pltpu.SMEM((t_total,), jnp.int32),
            pltpu.SemaphoreType.DMA,
            pltpu.SemaphoreType.DMA,
            pltpu.SemaphoreType.DMA,
        ],
        compiler_params=pltpu.CompilerParams(collective_id=0),
    )(ids, E)


# device time: 30591 ns/iter; 1.4689x vs baseline; 1.0014x over previous
import jax
import jax.numpy as jnp
from jax import lax
from jax.experimental import pallas as pl
from jax.experimental.pallas import tpu as pltpu


def kernel(ids, E):
    v_local, d = E.shape
    (t_total,) = ids.shape

    def body(ids_ref, e_ref, out_ref, tok_ref, row_ref,
             copy_sem, send_sem, recv_sem):
        x = lax.axis_index("x")
        y = lax.axis_index("y")
        z = lax.axis_index("z")
        partner = (1 - x, y, z)
        lo = x * v_local

        barrier_sem = pltpu.get_barrier_semaphore()
        pl.semaphore_signal(
            barrier_sem, inc=1,
            device_id=partner, device_id_type=pl.DeviceIdType.MESH,
        )
        pl.semaphore_wait(barrier_sem, 1)

        blk = 128
        n_blk = t_total // blk

        def comp(t, c):
            r = ids_ref[t] - lo
            mine = (r >= 0) & (r < v_local)
            tok_ref[c] = t
            row_ref[c] = r
            return c + mine.astype(jnp.int32)

        def issue(i, c):
            t = tok_ref[i]
            src = e_ref.at[pl.ds(row_ref[i], 1), :]
            dst = out_ref.at[pl.ds(t, 1), :]
            pltpu.make_async_remote_copy(
                src_ref=src,
                dst_ref=dst,
                send_sem=send_sem,
                recv_sem=recv_sem,
                device_id=partner,
                device_id_type=pl.DeviceIdType.MESH,
            ).start()
            pltpu.make_async_copy(src, dst, copy_sem).start()
            return c

        def block(b, c0):
            c1 = lax.fori_loop(b * blk, (b + 1) * blk, comp, c0)
            lax.fori_loop(c0, c1, issue, jnp.int32(0))
            return c1

        k = lax.fori_loop(0, n_blk, block, jnp.int32(0))

        def drain_mine(w):
            def f(_, c):
                pltpu.make_async_copy(
                    e_ref.at[pl.ds(0, w), :], out_ref.at[pl.ds(0, w), :],
                    copy_sem,
                ).wait()
                pltpu.make_async_remote_copy(
                    src_ref=e_ref.at[pl.ds(0, w), :],
                    dst_ref=out_ref.at[pl.ds(0, w), :],
                    send_sem=send_sem,
                    recv_sem=recv_sem,
                    device_id=partner,
                    device_id_type=pl.DeviceIdType.MESH,
                ).wait_send()
                return c
            return f

        def drain_recv(w):
            def f(_, c):
                pltpu.make_async_remote_copy(
                    src_ref=e_ref.at[pl.ds(0, w), :],
                    dst_ref=out_ref.at[pl.ds(0, w), :],
                    send_sem=send_sem,
                    recv_sem=recv_sem,
                    device_id=partner,
                    device_id_type=pl.DeviceIdType.MESH,
                ).wait_recv()
                return c
            return f

        n_recv = t_total - k
        lax.fori_loop(0, k // 8, drain_mine(8), 0)
        lax.fori_loop(0, k % 8, drain_mine(1), 0)
        lax.fori_loop(0, n_recv // 8, drain_recv(8), 0)
        lax.fori_loop(0, n_recv % 8, drain_recv(1), 0)

    return pl.pallas_call(
        body,
        out_shape=jax.ShapeDtypeStruct((t_total, d), jnp.float32),
        in_specs=[
            pl.BlockSpec(memory_space=pltpu.SMEM),
            pl.BlockSpec(memory_space=pl.ANY),
        ],
        out_specs=pl.BlockSpec(memory_space=pltpu.VMEM),
        scratch_shapes=[
            pltpu.SMEM((t_total,), jnp.int32),
            pltpu.SMEM((t_total,), jnp.int32),
            pltpu.SemaphoreType.DMA,
            pltpu.SemaphoreType.DMA,
            pltpu.SemaphoreType.DMA,
        ],
        compiler_params=pltpu.CompilerParams(collective_id=0),
    )(ids, E)
